# baseline (speedup 1.0000x reference)
import jax
import jax.numpy as jnp
from jax import lax
from jax.experimental import pallas as pl
from jax.experimental.pallas import tpu as pltpu, tpu_sc as plsc

NUM_Q = 100000
NUM_C = 1000
D = 128
MAX_C = 4
TOKENS = 51200

_info = plsc.get_sparse_core_info()
NC, NS, L = _info.num_cores, _info.num_subcores, _info.num_lanes
NW = NC * NS
PER_W = TOKENS // NW
CHUNK = 80
CHUNKS = PER_W // CHUNK
TG = CHUNK // 16


def _body(qseq, embq, embc, meta, out,
          qids_v, meta_v, cid0, cid1, cid2, cid3,
          qrows, c0, c1, c2, c3, fbuf, sem, sem2):
    wid = lax.axis_index("s") * NC + lax.axis_index("c")
    cidbufs = (cid0, cid1, cid2, cid3)
    crows = (c0, c1, c2, c3)

    def chunk_body(i, carry):
        base = wid * PER_W + i * CHUNK
        pltpu.sync_copy(qseq.at[pl.ds(base, CHUNK)], qids_v)
        pltpu.async_copy(meta.at[qids_v], meta_v, sem).wait()

        invs = []
        for tg in range(TG):
            toks = tg * 16 + lax.iota(jnp.int32, 16)
            ms = []
            for j in range(MAX_C):
                cid = plsc.load_gather(meta_v, [toks, jnp.full((16,), j, jnp.int32)])
                m = plsc.load_gather(meta_v, [toks, jnp.full((16,), MAX_C + j, jnp.int32)])
                ms.append(m)
                cidbufs[j][pl.ds(tg * 16, 16)] = jnp.where(m > 0, cid, NUM_C)
            cnt = ms[0] + ms[1] + ms[2] + ms[3]
            invs.append(1.0 / cnt.astype(jnp.float32))

        cps = [pltpu.async_copy(embq.at[qids_v], qrows, sem2)]
        for j in range(MAX_C):
            cps.append(pltpu.async_copy(embc.at[cidbufs[j]], crows[j], sem2))
        for cp in cps:
            cp.wait()

        def d_body(d, c):
            dvec = jnp.full((16,), d, jnp.int32)
            for tg in range(TG):
                toks = tg * 16 + lax.iota(jnp.int32, 16)
                acc = plsc.load_gather(c0, [toks, dvec])
                acc = acc + plsc.load_gather(c1, [toks, dvec])
                acc = acc + plsc.load_gather(c2, [toks, dvec])
                acc = acc + plsc.load_gather(c3, [toks, dvec])
                plsc.store_scatter(fbuf, [toks, dvec], acc * invs[tg])
            return c
        lax.fori_loop(0, D, d_body, 0)

        pltpu.sync_copy(fbuf, out.at[pl.ds(base, CHUNK), pl.ds(0, D)])
        pltpu.sync_copy(qrows, out.at[pl.ds(base, CHUNK), pl.ds(D, D)])
        return carry

    lax.fori_loop(0, CHUNKS, chunk_body, 0)


@jax.jit
def kernel(question_seq, embed_question, embed_concept, q2c_table, q2c_mask):
    B, S = question_seq.shape
    qseq = question_seq.reshape(-1).astype(jnp.int32)
    meta = jnp.concatenate(
        [q2c_table.astype(jnp.int32), q2c_mask.astype(jnp.int32),
         jnp.zeros((NUM_Q, 16 - 2 * MAX_C), jnp.int32)], axis=1)
    embc = jnp.concatenate(
        [embed_concept, jnp.zeros((8, D), jnp.float32)], axis=0)

    mesh = plsc.VectorSubcoreMesh(core_axis_name="c", subcore_axis_name="s")
    scratch = [
        pltpu.VMEM((CHUNK,), jnp.int32),
        pltpu.VMEM((CHUNK, 16), jnp.int32),
        pltpu.VMEM((CHUNK,), jnp.int32),
        pltpu.VMEM((CHUNK,), jnp.int32),
        pltpu.VMEM((CHUNK,), jnp.int32),
        pltpu.VMEM((CHUNK,), jnp.int32),
        pltpu.VMEM((CHUNK, D), jnp.float32),
        pltpu.VMEM((CHUNK, D), jnp.float32),
        pltpu.VMEM((CHUNK, D), jnp.float32),
        pltpu.VMEM((CHUNK, D), jnp.float32),
        pltpu.VMEM((CHUNK, D), jnp.float32),
        pltpu.VMEM((CHUNK, D), jnp.float32),
        pltpu.SemaphoreType.DMA,
        pltpu.SemaphoreType.DMA,
    ]
    out = pl.kernel(
        _body, mesh=mesh,
        out_type=jax.ShapeDtypeStruct((TOKENS, 2 * D), jnp.float32),
        scratch_types=scratch,
        compiler_params=pltpu.CompilerParams(
            needs_layout_passes=False, use_tc_tiling_on_sc=False),
    )(qseq, embed_question, embc, meta)
    return out.reshape(B, S, 2 * D)


# speedup vs baseline: 3.6586x; 3.6586x over previous
import jax
import jax.numpy as jnp
from jax import lax
from jax.experimental import pallas as pl
from jax.experimental.pallas import tpu as pltpu, tpu_sc as plsc

NUM_Q = 100000
NUM_C = 1000
D = 128
MAX_C = 4
TOKENS = 51200

_info = plsc.get_sparse_core_info()
NC, NS, L = _info.num_cores, _info.num_subcores, _info.num_lanes
NW = NC * NS
PER_W = TOKENS // NW
CHUNK = 80
CHUNKS = PER_W // CHUNK
TG = CHUNK // 16


def _body(qseq, embq, embc, meta, out,
          qids_v, meta_v, cid0, cid1, cid2, cid3,
          qrows, c0, c1, c2, c3, fbuf, sem, sem2):
    wid = lax.axis_index("s") * NC + lax.axis_index("c")
    cidbufs = (cid0, cid1, cid2, cid3)
    crows = (c0, c1, c2, c3)

    def chunk_body(i, carry):
        base = wid * PER_W + i * CHUNK
        pltpu.sync_copy(qseq.at[pl.ds(base, CHUNK)], qids_v)
        pltpu.async_copy(meta.at[qids_v], meta_v, sem).wait()

        wts = []
        for tg in range(TG):
            toks = tg * 16 + lax.iota(jnp.int32, 16)
            ms = []
            for j in range(MAX_C):
                cid = plsc.load_gather(meta_v, [toks, jnp.full((16,), j, jnp.int32)])
                m = plsc.load_gather(meta_v, [toks, jnp.full((16,), MAX_C + j, jnp.int32)])
                ms.append(m)
                cidbufs[j][pl.ds(tg * 16, 16)] = cid
            cnt = ms[0] + ms[1] + ms[2] + ms[3]
            inv = 1.0 / cnt.astype(jnp.float32)
            wts.append([m.astype(jnp.float32) * inv for m in ms])

        cps = [pltpu.async_copy(embq.at[qids_v], qrows, sem2)]
        for j in range(MAX_C):
            cps.append(pltpu.async_copy(embc.at[cidbufs[j]], crows[j], sem2))
        for cp in cps:
            cp.wait()

        def d_body(d, c):
            dvec = jnp.full((16,), d, jnp.int32)
            for tg in range(TG):
                toks = tg * 16 + lax.iota(jnp.int32, 16)
                w = wts[tg]
                acc = plsc.load_gather(c0, [toks, dvec]) * w[0]
                acc = acc + plsc.load_gather(c1, [toks, dvec]) * w[1]
                acc = acc + plsc.load_gather(c2, [toks, dvec]) * w[2]
                acc = acc + plsc.load_gather(c3, [toks, dvec]) * w[3]
                plsc.store_scatter(fbuf, [toks, dvec], acc)
            return c
        lax.fori_loop(0, D, d_body, 0)

        pltpu.sync_copy(fbuf, out.at[pl.ds(base, CHUNK), pl.ds(0, D)])
        pltpu.sync_copy(qrows, out.at[pl.ds(base, CHUNK), pl.ds(D, D)])
        return carry

    lax.fori_loop(0, CHUNKS, chunk_body, 0)


@jax.jit
def kernel(question_seq, embed_question, embed_concept, q2c_table, q2c_mask):
    B, S = question_seq.shape
    qseq = question_seq.reshape(-1).astype(jnp.int32)
    meta = jnp.concatenate(
        [q2c_table.astype(jnp.int32), q2c_mask.astype(jnp.int32),
         jnp.zeros((NUM_Q, 16 - 2 * MAX_C), jnp.int32)], axis=1)
    mesh = plsc.VectorSubcoreMesh(core_axis_name="c", subcore_axis_name="s")
    scratch = [
        pltpu.VMEM((CHUNK,), jnp.int32),
        pltpu.VMEM((CHUNK, 16), jnp.int32),
        pltpu.VMEM((CHUNK,), jnp.int32),
        pltpu.VMEM((CHUNK,), jnp.int32),
        pltpu.VMEM((CHUNK,), jnp.int32),
        pltpu.VMEM((CHUNK,), jnp.int32),
        pltpu.VMEM((CHUNK, D), jnp.float32),
        pltpu.VMEM((CHUNK, D), jnp.float32),
        pltpu.VMEM((CHUNK, D), jnp.float32),
        pltpu.VMEM((CHUNK, D), jnp.float32),
        pltpu.VMEM((CHUNK, D), jnp.float32),
        pltpu.VMEM((CHUNK, D), jnp.float32),
        pltpu.SemaphoreType.DMA,
        pltpu.SemaphoreType.DMA,
    ]
    out = pl.kernel(
        _body, mesh=mesh,
        out_type=jax.ShapeDtypeStruct((TOKENS, 2 * D), jnp.float32),
        scratch_types=scratch,
        compiler_params=pltpu.CompilerParams(
            needs_layout_passes=False, use_tc_tiling_on_sc=False),
    )(qseq, embed_question, embed_concept, meta)
    return out.reshape(B, S, 2 * D)


# speedup vs baseline: 4.0640x; 1.1108x over previous
import jax
import jax.numpy as jnp
from jax import lax
from jax.experimental import pallas as pl
from jax.experimental.pallas import tpu as pltpu, tpu_sc as plsc

NUM_Q = 100000
NUM_C = 1000
D = 128
MAX_C = 4
TOKENS = 51200

_info = plsc.get_sparse_core_info()
NC, NS, L = _info.num_cores, _info.num_subcores, _info.num_lanes
NW = NC * NS
PER_W = TOKENS // NW
CHUNK = 80
CHUNKS = PER_W // CHUNK
PAIRS = CHUNKS // 2
TG = CHUNK // 16


def _body(qseq, embq, embc, meta, out,
          qi0, qi1, mv0, mv1,
          ca0, ca1, ca2, ca3, cb0, cb1, cb2, cb3,
          qr0, qr1,
          c00, c01, c02, c03, c10, c11, c12, c13,
          wb0, wb1,
          sg0, sg1, sm0, sm1, ss0, ss1):
    wid = lax.axis_index("s") * NC + lax.axis_index("c")
    qids = (qi0, qi1)
    meta_v = (mv0, mv1)
    cidb = ((ca0, ca1, ca2, ca3), (cb0, cb1, cb2, cb3))
    qrows = (qr0, qr1)
    crows = ((c00, c01, c02, c03), (c10, c11, c12, c13))
    wbuf = (wb0, wb1)
    semG = (sg0, sg1)
    semM = (sm0, sm1)
    semS = (ss0, ss1)

    def copy_qids(ch, p):
        base = wid * PER_W + ch * CHUNK
        pltpu.sync_copy(qseq.at[pl.ds(base, CHUNK)], qids[p])

    def start_meta(p):
        return pltpu.async_copy(meta.at[qids[p]], meta_v[p], semM[p])

    def wait_meta(p):
        pltpu.make_async_copy(meta.at[qids[p]], meta_v[p], semM[p]).wait()

    def stage_meta(p):
        for tg in range(TG):
            toks = tg * 16 + lax.iota(jnp.int32, 16)
            ms = []
            for j in range(MAX_C):
                cid = plsc.load_gather(
                    meta_v[p], [toks, jnp.full((16,), j, jnp.int32)])
                m = plsc.load_gather(
                    meta_v[p], [toks, jnp.full((16,), MAX_C + j, jnp.int32)])
                ms.append(m)
                cidb[p][j][pl.ds(tg * 16, 16)] = cid
            cnt = ms[0] + ms[1] + ms[2] + ms[3]
            inv = 1.0 / cnt.astype(jnp.float32)
            for j in range(MAX_C):
                wbuf[p][j, pl.ds(tg * 16, 16)] = ms[j].astype(jnp.float32) * inv

    def start_gathers(p):
        pltpu.async_copy(embq.at[qids[p]], qrows[p], semG[p])
        for j in range(MAX_C):
            pltpu.async_copy(embc.at[cidb[p][j]], crows[p][j], semG[p])

    def wait_gathers(p):
        pltpu.make_async_copy(embq.at[qids[p]], qrows[p], semG[p]).wait()
        for j in range(MAX_C):
            pltpu.make_async_copy(embc.at[cidb[p][j]], crows[p][j], semG[p]).wait()

    def fusion(p):
        c0, c1, c2, c3 = crows[p]
        ws = []
        for tg in range(TG):
            ws.append([wbuf[p][j, pl.ds(tg * 16, 16)] for j in range(MAX_C)])

        def d_body(d, carry):
            for half in range(2):
                dvec = jnp.full((16,), 2 * d + half, jnp.int32)
                for tg in range(TG):
                    toks = tg * 16 + lax.iota(jnp.int32, 16)
                    w = ws[tg]
                    acc = plsc.load_gather(c0, [toks, dvec]) * w[0]
                    acc = acc + plsc.load_gather(c1, [toks, dvec]) * w[1]
                    acc = acc + plsc.load_gather(c2, [toks, dvec]) * w[2]
                    acc = acc + plsc.load_gather(c3, [toks, dvec]) * w[3]
                    plsc.store_scatter(c0, [toks, dvec], acc)
            return carry
        lax.fori_loop(0, D // 2, d_body, 0)

    def start_stores(ch, p):
        base = wid * PER_W + ch * CHUNK
        pltpu.async_copy(crows[p][0], out.at[pl.ds(base, CHUNK), pl.ds(0, D)],
                         semS[p])
        pltpu.async_copy(qrows[p], out.at[pl.ds(base, CHUNK), pl.ds(D, D)],
                         semS[p])

    def wait_stores(ch, p):
        base = wid * PER_W + ch * CHUNK
        pltpu.make_async_copy(
            crows[p][0], out.at[pl.ds(base, CHUNK), pl.ds(0, D)], semS[p]).wait()
        pltpu.make_async_copy(
            qrows[p], out.at[pl.ds(base, CHUNK), pl.ds(D, D)], semS[p]).wait()

    copy_qids(0, 0)
    start_meta(0)
    wait_meta(0)
    stage_meta(0)
    copy_qids(1, 1)
    start_meta(1)
    start_gathers(0)

    def pair_body(o, carry):
        for b in range(2):
            k = 2 * o + b
            wait_gathers(b)
            if b == 0:
                wait_meta(1)
                stage_meta(1)
            else:
                @pl.when(o < PAIRS - 1)
                def _():
                    wait_meta(0)
                    stage_meta(0)
            @pl.when(o < PAIRS - 1)
            def _():
                copy_qids(k + 2, b)
                start_meta(b)
            if b == 0:
                @pl.when(o > 0)
                def _():
                    wait_stores(k - 1, 1)
                start_gathers(1)
            else:
                wait_stores(k - 1, 0)

                @pl.when(o < PAIRS - 1)
                def _():
                    start_gathers(0)
            fusion(b)
            start_stores(k, b)
        return carry

    lax.fori_loop(0, PAIRS, pair_body, 0)
    wait_stores(CHUNKS - 1, 1)


@jax.jit
def kernel(question_seq, embed_question, embed_concept, q2c_table, q2c_mask):
    B, S = question_seq.shape
    qseq = question_seq.reshape(-1).astype(jnp.int32)
    meta = jnp.concatenate(
        [q2c_table.astype(jnp.int32), q2c_mask.astype(jnp.int32),
         jnp.zeros((NUM_Q, 16 - 2 * MAX_C), jnp.int32)], axis=1)

    mesh = plsc.VectorSubcoreMesh(core_axis_name="c", subcore_axis_name="s")
    fv = jnp.float32
    iv = jnp.int32
    scratch = (
        [pltpu.VMEM((CHUNK,), iv)] * 2 +
        [pltpu.VMEM((CHUNK, 16), iv)] * 2 +
        [pltpu.VMEM((CHUNK,), iv)] * 8 +
        [pltpu.VMEM((CHUNK, D), fv)] * 2 +
        [pltpu.VMEM((CHUNK, D), fv)] * 8 +
        [pltpu.VMEM((MAX_C, CHUNK), fv)] * 2 +
        [pltpu.SemaphoreType.DMA] * 6
    )
    out = pl.kernel(
        _body, mesh=mesh,
        out_type=jax.ShapeDtypeStruct((TOKENS, 2 * D), jnp.float32),
        scratch_types=scratch,
        compiler_params=pltpu.CompilerParams(
            needs_layout_passes=False, use_tc_tiling_on_sc=False),
    )(qseq, embed_question, embed_concept, meta)
    return out.reshape(B, S, 2 * D)


# speedup vs baseline: 4.9724x; 1.2235x over previous
import jax
import jax.numpy as jnp
from jax import lax
from jax.experimental import pallas as pl
from jax.experimental.pallas import tpu as pltpu, tpu_sc as plsc

NUM_Q = 100000
NUM_C = 1000
D = 128
MAX_C = 4
TOKENS = 51200

_info = plsc.get_sparse_core_info()
NC, NS, L = _info.num_cores, _info.num_subcores, _info.num_lanes
NW = NC * NS
PER_W = TOKENS // NW
CHUNK = 80
CHUNKS = PER_W // CHUNK
PAIRS = CHUNKS // 2
TG = CHUNK // 16


def _body(qseq, embq, embc, meta, out,
          qi0, qi1, mv0, mv1,
          ca0, ca1, ca2, ca3, cb0, cb1, cb2, cb3,
          qr0, qr1,
          c00, c01, c02, c03, c10, c11, c12, c13,
          wb0, wb1,
          sg0, sg1, sm0, sm1, ss0, ss1):
    wid = lax.axis_index("s") * NC + lax.axis_index("c")
    qids = (qi0, qi1)
    meta_v = (mv0, mv1)
    cidb = ((ca0, ca1, ca2, ca3), (cb0, cb1, cb2, cb3))
    qrows = (qr0, qr1)
    crows = ((c00, c01, c02, c03), (c10, c11, c12, c13))
    wbuf = (wb0, wb1)
    semG = (sg0, sg1)
    semM = (sm0, sm1)
    semS = (ss0, ss1)

    def copy_qids(ch, p):
        base = wid * PER_W + ch * CHUNK
        pltpu.sync_copy(qseq.at[pl.ds(base, CHUNK)], qids[p])

    def start_meta(p):
        return pltpu.async_copy(meta.at[qids[p]], meta_v[p], semM[p])

    def wait_meta(p):
        pltpu.make_async_copy(meta.at[qids[p]], meta_v[p], semM[p]).wait()

    def stage_meta(p):
        for tg in range(TG):
            toks = tg * 16 + lax.iota(jnp.int32, 16)
            ms = []
            for j in range(MAX_C):
                cid = plsc.load_gather(
                    meta_v[p], [toks, jnp.full((16,), j, jnp.int32)])
                m = plsc.load_gather(
                    meta_v[p], [toks, jnp.full((16,), MAX_C + j, jnp.int32)])
                ms.append(m)
                cidb[p][j][pl.ds(tg * 16, 16)] = cid
            cnt = ms[0] + ms[1] + ms[2] + ms[3]
            inv = 1.0 / cnt.astype(jnp.float32)
            for j in range(MAX_C):
                wbuf[p][j, pl.ds(tg * 16, 16)] = ms[j].astype(jnp.float32) * inv

    def start_gathers(p):
        pltpu.async_copy(embq.at[qids[p]], qrows[p], semG[p])
        for j in range(MAX_C):
            pltpu.async_copy(embc.at[cidb[p][j]], crows[p][j], semG[p])

    def wait_gathers(p):
        pltpu.make_async_copy(embq.at[qids[p]], qrows[p], semG[p]).wait()
        for j in range(MAX_C):
            pltpu.make_async_copy(embc.at[cidb[p][j]], crows[p][j], semG[p]).wait()

    def fusion(p):
        c0, c1, c2, c3 = crows[p]
        ws = []
        for tg in range(TG):
            ws.append([wbuf[p][j, pl.ds(tg * 16, 16)] for j in range(MAX_C)])

        @plsc.parallel_loop(0, D, unroll=4)
        def d_body(d):
            dvec = jnp.full((16,), d, jnp.int32)
            for tg in range(TG):
                toks = tg * 16 + lax.iota(jnp.int32, 16)
                w = ws[tg]
                acc = plsc.load_gather(c0, [toks, dvec]) * w[0]
                acc = acc + plsc.load_gather(c1, [toks, dvec]) * w[1]
                acc = acc + plsc.load_gather(c2, [toks, dvec]) * w[2]
                acc = acc + plsc.load_gather(c3, [toks, dvec]) * w[3]
                plsc.store_scatter(c0, [toks, dvec], acc)

    def start_stores(ch, p):
        base = wid * PER_W + ch * CHUNK
        pltpu.async_copy(crows[p][0], out.at[pl.ds(base, CHUNK), pl.ds(0, D)],
                         semS[p])
        pltpu.async_copy(qrows[p], out.at[pl.ds(base, CHUNK), pl.ds(D, D)],
                         semS[p])

    def wait_stores(ch, p):
        base = wid * PER_W + ch * CHUNK
        pltpu.make_async_copy(
            crows[p][0], out.at[pl.ds(base, CHUNK), pl.ds(0, D)], semS[p]).wait()
        pltpu.make_async_copy(
            qrows[p], out.at[pl.ds(base, CHUNK), pl.ds(D, D)], semS[p]).wait()

    copy_qids(0, 0)
    start_meta(0)
    wait_meta(0)
    stage_meta(0)
    copy_qids(1, 1)
    start_meta(1)
    start_gathers(0)

    def pair_body(o, carry):
        for b in range(2):
            k = 2 * o + b
            wait_gathers(b)
            if b == 0:
                wait_meta(1)
                stage_meta(1)
            else:
                @pl.when(o < PAIRS - 1)
                def _():
                    wait_meta(0)
                    stage_meta(0)
            @pl.when(o < PAIRS - 1)
            def _():
                copy_qids(k + 2, b)
                start_meta(b)
            if b == 0:
                @pl.when(o > 0)
                def _():
                    wait_stores(k - 1, 1)
                start_gathers(1)
            else:
                wait_stores(k - 1, 0)

                @pl.when(o < PAIRS - 1)
                def _():
                    start_gathers(0)
            fusion(b)
            start_stores(k, b)
        return carry

    lax.fori_loop(0, PAIRS, pair_body, 0)
    wait_stores(CHUNKS - 1, 1)


@jax.jit
def kernel(question_seq, embed_question, embed_concept, q2c_table, q2c_mask):
    B, S = question_seq.shape
    qseq = question_seq.reshape(-1).astype(jnp.int32)
    meta = jnp.concatenate(
        [q2c_table.astype(jnp.int32), q2c_mask.astype(jnp.int32),
         jnp.zeros((NUM_Q, 16 - 2 * MAX_C), jnp.int32)], axis=1)

    mesh = plsc.VectorSubcoreMesh(core_axis_name="c", subcore_axis_name="s")
    fv = jnp.float32
    iv = jnp.int32
    scratch = (
        [pltpu.VMEM((CHUNK,), iv)] * 2 +
        [pltpu.VMEM((CHUNK, 16), iv)] * 2 +
        [pltpu.VMEM((CHUNK,), iv)] * 8 +
        [pltpu.VMEM((CHUNK, D), fv)] * 2 +
        [pltpu.VMEM((CHUNK, D), fv)] * 8 +
        [pltpu.VMEM((MAX_C, CHUNK), fv)] * 2 +
        [pltpu.SemaphoreType.DMA] * 6
    )
    out = pl.kernel(
        _body, mesh=mesh,
        out_type=jax.ShapeDtypeStruct((TOKENS, 2 * D), jnp.float32),
        scratch_types=scratch,
        compiler_params=pltpu.CompilerParams(
            needs_layout_passes=False, use_tc_tiling_on_sc=False),
    )(qseq, embed_question, embed_concept, meta)
    return out.reshape(B, S, 2 * D)


# speedup vs baseline: 7.2997x; 1.4680x over previous
import jax
import jax.numpy as jnp
from jax import lax
from jax.experimental import pallas as pl
from jax.experimental.pallas import tpu as pltpu, tpu_sc as plsc

NUM_Q = 100000
NUM_C = 1000
D = 128
MAX_C = 4
TOKENS = 51200

_info = plsc.get_sparse_core_info()
NC, NS, L = _info.num_cores, _info.num_subcores, _info.num_lanes
NW = NC * NS
PER_W = TOKENS // NW
CHUNK = 80
CHUNKS = PER_W // CHUNK
PAIRS = CHUNKS // 2
TG = CHUNK // 16


def _body(qseq, embq, ctab, meta, out,
          qi0, qi1, mv0, mv1,
          ca0, ca1, ca2, ca3, cb0, cb1, cb2, cb3,
          qr0, qr1, fb0, fb1, ctab_v,
          wb0, wb1,
          sg0, sg1, sm0, sm1, ss0, ss1):
    wid = lax.axis_index("s") * NC + lax.axis_index("c")
    qids = (qi0, qi1)
    meta_v = (mv0, mv1)
    cidb = ((ca0, ca1, ca2, ca3), (cb0, cb1, cb2, cb3))
    qrows = (qr0, qr1)
    fbuf = (fb0, fb1)
    wbuf = (wb0, wb1)
    semG = (sg0, sg1)
    semM = (sm0, sm1)
    semS = (ss0, ss1)

    pltpu.sync_copy(ctab, ctab_v)

    def copy_qids(ch, p):
        base = wid * PER_W + ch * CHUNK
        pltpu.sync_copy(qseq.at[pl.ds(base, CHUNK)], qids[p])

    def start_meta(p):
        return pltpu.async_copy(meta.at[qids[p]], meta_v[p], semM[p])

    def wait_meta(p):
        pltpu.make_async_copy(meta.at[qids[p]], meta_v[p], semM[p]).wait()

    def stage_meta(p):
        for tg in range(TG):
            toks = tg * 16 + lax.iota(jnp.int32, 16)
            ms = []
            for j in range(MAX_C):
                cid = plsc.load_gather(
                    meta_v[p], [toks, jnp.full((16,), j, jnp.int32)])
                m = plsc.load_gather(
                    meta_v[p], [toks, jnp.full((16,), MAX_C + j, jnp.int32)])
                ms.append(m)
                cidb[p][j][pl.ds(tg * 16, 16)] = cid
            cnt = ms[0] + ms[1] + ms[2] + ms[3]
            inv = 1.0 / cnt.astype(jnp.float32)
            for j in range(MAX_C):
                wbuf[p][j, pl.ds(tg * 16, 16)] = ms[j].astype(jnp.float32) * inv

    def start_gathers(p):
        pltpu.async_copy(embq.at[qids[p]], qrows[p], semG[p])

    def wait_gathers(p):
        pltpu.make_async_copy(embq.at[qids[p]], qrows[p], semG[p]).wait()

    def fusion(p):
        def unpack2(word):
            lo = plsc.bitcast(lax.shift_left(word, 16), jnp.float32)
            hi = plsc.bitcast(
                lax.bitwise_and(word, jnp.int32(-65536)), jnp.float32)
            return lo, hi

        for tg in range(TG):
            toks = tg * 16 + lax.iota(jnp.int32, 16)
            cv = [cidb[p][j][pl.ds(tg * 16, 16)] for j in range(MAX_C)]
            ws = [wbuf[p][j, pl.ds(tg * 16, 16)] for j in range(MAX_C)]

            @plsc.parallel_loop(0, D // 2, unroll=4)
            def dp_body(dp):
                dvec = jnp.full((16,), dp, jnp.int32)
                lo0, hi0 = unpack2(plsc.load_gather(ctab_v, [cv[0], dvec]))
                lo1, hi1 = unpack2(plsc.load_gather(ctab_v, [cv[1], dvec]))
                lo2, hi2 = unpack2(plsc.load_gather(ctab_v, [cv[2], dvec]))
                lo3, hi3 = unpack2(plsc.load_gather(ctab_v, [cv[3], dvec]))
                acc_lo = lo0 * ws[0] + lo1 * ws[1] + lo2 * ws[2] + lo3 * ws[3]
                acc_hi = hi0 * ws[0] + hi1 * ws[1] + hi2 * ws[2] + hi3 * ws[3]
                plsc.store_scatter(fbuf[p], [toks, 2 * dvec], acc_lo)
                plsc.store_scatter(fbuf[p], [toks, 2 * dvec + 1], acc_hi)

    def start_stores(ch, p):
        base = wid * PER_W + ch * CHUNK
        pltpu.async_copy(fbuf[p], out.at[pl.ds(base, CHUNK), pl.ds(0, D)],
                         semS[p])
        pltpu.async_copy(qrows[p], out.at[pl.ds(base, CHUNK), pl.ds(D, D)],
                         semS[p])

    def wait_stores(ch, p):
        base = wid * PER_W + ch * CHUNK
        pltpu.make_async_copy(
            fbuf[p], out.at[pl.ds(base, CHUNK), pl.ds(0, D)], semS[p]).wait()
        pltpu.make_async_copy(
            qrows[p], out.at[pl.ds(base, CHUNK), pl.ds(D, D)], semS[p]).wait()

    copy_qids(0, 0)
    start_meta(0)
    wait_meta(0)
    stage_meta(0)
    copy_qids(1, 1)
    start_meta(1)
    start_gathers(0)

    def pair_body(o, carry):
        for b in range(2):
            k = 2 * o + b
            wait_gathers(b)
            if b == 0:
                wait_meta(1)
                stage_meta(1)
            else:
                @pl.when(o < PAIRS - 1)
                def _():
                    wait_meta(0)
                    stage_meta(0)
            @pl.when(o < PAIRS - 1)
            def _():
                copy_qids(k + 2, b)
                start_meta(b)
            if b == 0:
                @pl.when(o > 0)
                def _():
                    wait_stores(k - 1, 1)
                start_gathers(1)
            else:
                wait_stores(k - 1, 0)

                @pl.when(o < PAIRS - 1)
                def _():
                    start_gathers(0)
            fusion(b)
            start_stores(k, b)
        return carry

    lax.fori_loop(0, PAIRS, pair_body, 0)
    wait_stores(CHUNKS - 1, 1)


@jax.jit
def kernel(question_seq, embed_question, embed_concept, q2c_table, q2c_mask):
    B, S = question_seq.shape
    qseq = question_seq.reshape(-1).astype(jnp.int32)
    meta = jnp.concatenate(
        [q2c_table.astype(jnp.int32), q2c_mask.astype(jnp.int32),
         jnp.zeros((NUM_Q, 16 - 2 * MAX_C), jnp.int32)], axis=1)
    ctab = jax.lax.bitcast_convert_type(
        embed_concept.astype(jnp.bfloat16).reshape(NUM_C, D // 2, 2),
        jnp.int32)

    mesh = plsc.VectorSubcoreMesh(core_axis_name="c", subcore_axis_name="s")
    fv = jnp.float32
    iv = jnp.int32
    scratch = (
        [pltpu.VMEM((CHUNK,), iv)] * 2 +
        [pltpu.VMEM((CHUNK, 16), iv)] * 2 +
        [pltpu.VMEM((CHUNK,), iv)] * 8 +
        [pltpu.VMEM((CHUNK, D), fv)] * 2 +
        [pltpu.VMEM((CHUNK, D), fv)] * 2 +
        [pltpu.VMEM((NUM_C, D // 2), iv)] * 1 +
        [pltpu.VMEM((MAX_C, CHUNK), fv)] * 2 +
        [pltpu.SemaphoreType.DMA] * 6
    )
    out = pl.kernel(
        _body, mesh=mesh,
        out_type=jax.ShapeDtypeStruct((TOKENS, 2 * D), jnp.float32),
        scratch_types=scratch,
        compiler_params=pltpu.CompilerParams(
            needs_layout_passes=False, use_tc_tiling_on_sc=False),
    )(qseq, embed_question, ctab, meta)
    return out.reshape(B, S, 2 * D)


# speedup vs baseline: 7.3350x; 1.0048x over previous
import jax
import jax.numpy as jnp
from jax import lax
from jax.experimental import pallas as pl
from jax.experimental.pallas import tpu as pltpu, tpu_sc as plsc

NUM_Q = 100000
NUM_C = 1000
D = 128
MAX_C = 4
TOKENS = 51200

_info = plsc.get_sparse_core_info()
NC, NS, L = _info.num_cores, _info.num_subcores, _info.num_lanes
NW = NC * NS
PER_W = TOKENS // NW
CHUNK = 80
CHUNKS = PER_W // CHUNK
PAIRS = CHUNKS // 2
TG = CHUNK // 16


def _body(qseq, embq, ctab, meta, out,
          qi0, qi1, mv0, mv1,
          ca0, ca1, ca2, ca3, cb0, cb1, cb2, cb3,
          qr0, qr1, fb0, fb1, ctab_v,
          wb0, wb1,
          sg0, sg1, sm0, sm1, ss0, ss1):
    wid = lax.axis_index("s") * NC + lax.axis_index("c")
    qids = (qi0, qi1)
    meta_v = (mv0, mv1)
    cidb = ((ca0, ca1, ca2, ca3), (cb0, cb1, cb2, cb3))
    qrows = (qr0, qr1)
    fbuf = (fb0, fb1)
    wbuf = (wb0, wb1)
    semG = (sg0, sg1)
    semM = (sm0, sm1)
    semS = (ss0, ss1)

    pltpu.sync_copy(ctab, ctab_v)

    def copy_qids(ch, p):
        base = wid * PER_W + ch * CHUNK
        pltpu.sync_copy(qseq.at[pl.ds(base, CHUNK)], qids[p])

    def start_meta(p):
        return pltpu.async_copy(meta.at[qids[p]], meta_v[p], semM[p])

    def wait_meta(p):
        pltpu.make_async_copy(meta.at[qids[p]], meta_v[p], semM[p]).wait()

    def stage_meta(p):
        for tg in range(TG):
            toks = tg * 16 + lax.iota(jnp.int32, 16)
            ms = []
            for j in range(MAX_C):
                cid = plsc.load_gather(
                    meta_v[p], [toks, jnp.full((16,), j, jnp.int32)])
                m = plsc.load_gather(
                    meta_v[p], [toks, jnp.full((16,), MAX_C + j, jnp.int32)])
                ms.append(m)
                cidb[p][j][pl.ds(tg * 16, 16)] = cid
            cnt = ms[0] + ms[1] + ms[2] + ms[3]
            inv = 1.0 / cnt.astype(jnp.float32)
            for j in range(MAX_C):
                wbuf[p][j, pl.ds(tg * 16, 16)] = ms[j].astype(jnp.float32) * inv

    def start_gathers(p):
        pltpu.async_copy(embq.at[qids[p]], qrows[p], semG[p])

    def wait_gathers(p):
        pltpu.make_async_copy(embq.at[qids[p]], qrows[p], semG[p]).wait()

    def fusion(p):
        def unpack2(word):
            lo = plsc.bitcast(lax.shift_left(word, 16), jnp.float32)
            hi = plsc.bitcast(
                lax.bitwise_and(word, jnp.int32(-65536)), jnp.float32)
            return lo, hi

        for tg in range(TG):
            toks = tg * 16 + lax.iota(jnp.int32, 16)
            cv = [cidb[p][j][pl.ds(tg * 16, 16)] for j in range(MAX_C)]
            ws = [wbuf[p][j, pl.ds(tg * 16, 16)] for j in range(MAX_C)]

            @plsc.parallel_loop(0, D // 2, unroll=4)
            def dp_body(dp):
                dvec = jnp.full((16,), dp, jnp.int32)
                lo0, hi0 = unpack2(plsc.load_gather(ctab_v, [cv[0], dvec]))
                lo1, hi1 = unpack2(plsc.load_gather(ctab_v, [cv[1], dvec]))
                lo2, hi2 = unpack2(plsc.load_gather(ctab_v, [cv[2], dvec]))
                lo3, hi3 = unpack2(plsc.load_gather(ctab_v, [cv[3], dvec]))
                acc_lo = lo0 * ws[0] + lo1 * ws[1] + lo2 * ws[2] + lo3 * ws[3]
                acc_hi = hi0 * ws[0] + hi1 * ws[1] + hi2 * ws[2] + hi3 * ws[3]
                plsc.store_scatter(fbuf[p], [toks, 2 * dvec], acc_lo)
                plsc.store_scatter(fbuf[p], [toks, 2 * dvec + 1], acc_hi)

    def start_stores(ch, p):
        base = wid * PER_W + ch * CHUNK
        pltpu.async_copy(fbuf[p], out.at[pl.ds(base, CHUNK), pl.ds(0, D)],
                         semS[p])
        pltpu.async_copy(qrows[p], out.at[pl.ds(base, CHUNK), pl.ds(D, D)],
                         semS[p])

    def wait_stores(ch, p):
        base = wid * PER_W + ch * CHUNK
        pltpu.make_async_copy(
            fbuf[p], out.at[pl.ds(base, CHUNK), pl.ds(0, D)], semS[p]).wait()
        pltpu.make_async_copy(
            qrows[p], out.at[pl.ds(base, CHUNK), pl.ds(D, D)], semS[p]).wait()

    copy_qids(0, 0)
    start_meta(0)
    wait_meta(0)
    stage_meta(0)
    copy_qids(1, 1)
    start_meta(1)
    start_gathers(0)

    def pair_body(o, carry):
        for b in range(2):
            k = 2 * o + b
            with jax.named_scope("wgath"):
                wait_gathers(b)
            if b == 0:
                with jax.named_scope("smeta"):
                    wait_meta(1)
                    stage_meta(1)
            else:
                @pl.when(o < PAIRS - 1)
                def _():
                    wait_meta(0)
                    stage_meta(0)
            @pl.when(o < PAIRS - 1)
            def _():
                copy_qids(k + 2, b)
                start_meta(b)
            if b == 0:
                with jax.named_scope("wstore"):
                    @pl.when(o > 0)
                    def _():
                        wait_stores(k - 1, 1)
                start_gathers(1)
            else:
                with jax.named_scope("wstore"):
                    wait_stores(k - 1, 0)

                @pl.when(o < PAIRS - 1)
                def _():
                    start_gathers(0)
            with jax.named_scope("fus"):
                fusion(b)
            start_stores(k, b)
        return carry

    lax.fori_loop(0, PAIRS, pair_body, 0)
    wait_stores(CHUNKS - 1, 1)


@jax.jit
def kernel(question_seq, embed_question, embed_concept, q2c_table, q2c_mask):
    B, S = question_seq.shape
    qseq = question_seq.reshape(-1).astype(jnp.int32)
    meta = jnp.concatenate(
        [q2c_table.astype(jnp.int32), q2c_mask.astype(jnp.int32),
         jnp.zeros((NUM_Q, 16 - 2 * MAX_C), jnp.int32)], axis=1)
    ctab = jax.lax.bitcast_convert_type(
        embed_concept.astype(jnp.bfloat16).reshape(NUM_C, D // 2, 2),
        jnp.int32)

    mesh = plsc.VectorSubcoreMesh(core_axis_name="c", subcore_axis_name="s")
    fv = jnp.float32
    iv = jnp.int32
    scratch = (
        [pltpu.VMEM((CHUNK,), iv)] * 2 +
        [pltpu.VMEM((CHUNK, 16), iv)] * 2 +
        [pltpu.VMEM((CHUNK,), iv)] * 8 +
        [pltpu.VMEM((CHUNK, D), fv)] * 2 +
        [pltpu.VMEM((CHUNK, D), fv)] * 2 +
        [pltpu.VMEM((NUM_C, D // 2), iv)] * 1 +
        [pltpu.VMEM((MAX_C, CHUNK), fv)] * 2 +
        [pltpu.SemaphoreType.DMA] * 6
    )
    out = pl.kernel(
        _body, mesh=mesh,
        out_type=jax.ShapeDtypeStruct((TOKENS, 2 * D), jnp.float32),
        scratch_types=scratch,
        compiler_params=pltpu.CompilerParams(
            needs_layout_passes=False, use_tc_tiling_on_sc=False),
    )(qseq, embed_question, ctab, meta)
    return out.reshape(B, S, 2 * D)
